# baseline (speedup 1.0000x reference)
import jax
import jax.numpy as jnp
from jax import lax
from jax.experimental import pallas as pl
from jax.experimental.pallas import tpu as pltpu
from jax.experimental.pallas import tpu_sc as plsc

NC = 2
NS = 16
NW = NC * NS
V = 20
D = 80
B = 64 * 4096
BPW = B // NW
C = 256
NCHUNK = BPW // C


def _emb_kernel(idx_hbm, table_hbm, out_hbm, table_v, idx_v, rows_v, osems):
    wid = lax.axis_index("s") * NC + lax.axis_index("c")
    pltpu.sync_copy(table_hbm, table_v)
    for rr in range(2):
        pltpu.sync_copy(idx_hbm.at[2 * wid + rr, :],
                        idx_v.at[rr, pl.ds(0, 4096)])

    def chunk_pos(cid):
        return cid // (4096 // C), (cid % (4096 // C)) * C

    def compute_chunk(cid, b):
        r, col = chunk_pos(cid)

        @plsc.parallel_loop(0, C // 16, step=1, unroll=1)
        def group_body(g):
            idxv = idx_v[r, pl.ds(col + g * 16, 16)]
            for l in range(16):
                src = idxv[l]
                dst = g * 16 + l
                for j in range(D // 16):
                    rows_v[b, dst, pl.ds(16 * j, 16)] = (
                        table_v[src, pl.ds(16 * j, 16)])

    def outer(cid, carry):
        b = lax.rem(cid, 2)

        @pl.when(cid > 1)
        def _wait():
            pltpu.make_async_copy(rows_v.at[b],
                                  out_hbm.at[2 * wid, pl.ds(0, C)],
                                  osems.at[b]).wait()

        compute_chunk(cid, b)
        r, col = chunk_pos(cid)
        pltpu.make_async_copy(rows_v.at[b],
                              out_hbm.at[2 * wid + r, pl.ds(col, C)],
                              osems.at[b]).start()
        return carry

    lax.fori_loop(0, NCHUNK, outer, 0)

    for b in range(2):
        pltpu.make_async_copy(rows_v.at[b], out_hbm.at[2 * wid, pl.ds(0, C)],
                              osems.at[b]).wait()


@jax.jit
def _emb(atom_types, table):
    mesh = plsc.VectorSubcoreMesh(core_axis_name="c", subcore_axis_name="s")
    run = pl.kernel(
        _emb_kernel,
        out_type=jax.ShapeDtypeStruct((64, 4096, D), jnp.float32),
        mesh=mesh,
        scratch_types=[
            pltpu.VMEM((V, D), jnp.float32),
            pltpu.VMEM((2, 4104), jnp.int32),
            pltpu.VMEM((2, C, D), jnp.float32),
            pltpu.SemaphoreType.DMA((2,)),
        ],
    )
    return run(atom_types, table)


def kernel(atom_types, table):
    return _emb(atom_types.astype(jnp.int32), table)


# speedup vs baseline: 1.0033x; 1.0033x over previous
import jax
import jax.numpy as jnp
from jax import lax
from jax.experimental import pallas as pl
from jax.experimental.pallas import tpu as pltpu
from jax.experimental.pallas import tpu_sc as plsc

NC = 2
NS = 16
NW = NC * NS
V = 20
D = 80
B = 64 * 4096
BPW = B // NW
C = 256
NCHUNK = BPW // C


def _emb_kernel(idx_hbm, table_hbm, out_hbm, table_v, idx_v, rows_v, osems):
    wid = lax.axis_index("s") * NC + lax.axis_index("c")
    pltpu.sync_copy(table_hbm, table_v)
    pltpu.sync_copy(idx_hbm.at[pl.ds(2 * wid, 2), :], idx_v)

    def chunk_pos(cid):
        return cid // (4096 // C), (cid % (4096 // C)) * C

    def compute_chunk(cid, b):
        r, col = chunk_pos(cid)

        @plsc.parallel_loop(0, C // 16, step=1, unroll=1)
        def group_body(g):
            idxv = idx_v[r, pl.ds(col + g * 16, 16)]
            for l in range(16):
                src = idxv[l]
                dst = g * 16 + l
                for j in range(D // 16):
                    rows_v[b, dst, pl.ds(16 * j, 16)] = (
                        table_v[src, pl.ds(16 * j, 16)])

    def outer(cid, carry):
        b = lax.rem(cid, 2)

        @pl.when(cid > 1)
        def _wait():
            pltpu.make_async_copy(rows_v.at[b],
                                  out_hbm.at[2 * wid, pl.ds(0, C)],
                                  osems.at[b]).wait()

        compute_chunk(cid, b)
        r, col = chunk_pos(cid)
        pltpu.make_async_copy(rows_v.at[b],
                              out_hbm.at[2 * wid + r, pl.ds(col, C)],
                              osems.at[b]).start()
        return carry

    lax.fori_loop(0, NCHUNK, outer, 0)

    for b in range(2):
        pltpu.make_async_copy(rows_v.at[b], out_hbm.at[2 * wid, pl.ds(0, C)],
                              osems.at[b]).wait()


@jax.jit
def _emb(atom_types, table):
    mesh = plsc.VectorSubcoreMesh(core_axis_name="c", subcore_axis_name="s")
    run = pl.kernel(
        _emb_kernel,
        out_type=jax.ShapeDtypeStruct((64, 4096, D), jnp.float32),
        mesh=mesh,
        scratch_types=[
            pltpu.VMEM((V, D), jnp.float32),
            pltpu.VMEM((2, 4096), jnp.int32),
            pltpu.VMEM((2, C, D), jnp.float32),
            pltpu.SemaphoreType.DMA((2,)),
        ],
    )
    return run(atom_types, table)


def kernel(atom_types, table):
    return _emb(atom_types.astype(jnp.int32), table)
